# Initial kernel scaffold; baseline (speedup 1.0000x reference)
import jax
import jax.numpy as jnp
from jax.experimental import pallas as pl


def kernel(x, table, bn_gamma, bn_beta, ctrl_W, ctrl_b, ctrl_bn_g, ctrl_bn_b, cross_W, cross_b, mlp_W1, mlp_b1, bn1_g, bn1_b, mlp_W2, mlp_b2, bn2_g, bn2_b, lin_W, lin_b):
    raise NotImplementedError("write your pallas kernel here")


# speedup vs baseline: 1.1248x; 1.1248x over previous
import functools

import jax
import jax.numpy as jnp
from jax import lax
from jax.experimental import pallas as pl
from jax.experimental.pallas import tpu as pltpu
from jax.experimental.pallas import tpu_sc as plsc

_B = 16384
_F = 26
_VOCAB = 100000
_D = 16
_ED = _F * _D
_K = 13
_EPS = 1e-5
_ROWS = _B * _F

_NC, _NS = 2, 16
_NW = _NC * _NS
_RPW = _ROWS // _NW
_G = 128
_CH = 1024
_GPC = _CH // _G
_NCH = _RPW // _CH

_BS = 1024
_NB = _B // _BS


def _gather_body(table_hbm, idx_hbm, out_hbm, idx0, idx1, rows0, rows1,
                 sem0, sem1):
    wid = lax.axis_index("s") * _NC + lax.axis_index("c")
    base = pl.multiple_of(wid * _RPW, 8)
    grp_base = wid * (_RPW // _G)
    idx_b = (idx0, idx1)
    row_b = (rows0, rows1)
    sem_b = (sem0, sem1)
    copies = [None] * _NCH

    def start(c):
        b = c % 2
        pltpu.sync_copy(idx_hbm.at[pl.ds(grp_base + c * _GPC, _GPC)], idx_b[b])
        cps = []
        for j in range(_GPC):
            cps.append(pltpu.async_copy(
                table_hbm.at[idx_b[b].at[j]],
                row_b[b].at[pl.ds(j * _G, _G)],
                sem_b[b]))
        copies[c] = cps

    start(0)
    for c in range(_NCH):
        if c + 1 < _NCH:
            start(c + 1)
        for cp in copies[c]:
            cp.wait()
        off = pl.multiple_of(base + c * _CH, 8)
        pltpu.sync_copy(row_b[c % 2], out_hbm.at[pl.ds(off, _CH)])


@functools.cache
def _gather_kernel_fn():
    mesh = plsc.VectorSubcoreMesh(core_axis_name="c", subcore_axis_name="s")
    return pl.kernel(
        _gather_body,
        out_type=jax.ShapeDtypeStruct((_ROWS, _D), jnp.float32),
        mesh=mesh,
        scratch_types=[
            pltpu.VMEM((_GPC, _G), jnp.int32),
            pltpu.VMEM((_GPC, _G), jnp.int32),
            pltpu.VMEM((_CH, _D), jnp.float32),
            pltpu.VMEM((_CH, _D), jnp.float32),
            pltpu.SemaphoreType.DMA,
            pltpu.SemaphoreType.DMA,
        ],
        compiler_params=pltpu.CompilerParams(use_tc_tiling_on_sc=False),
    )


def _gather_kernel(table, idx2):
    return _gather_kernel_fn()(table, idx2)


def _field_stats_body(e_ref, acc_ref):
    i = pl.program_id(0)
    blk = e_ref[...]
    s1 = jnp.sum(blk, axis=0, keepdims=True)
    s2 = jnp.sum(blk * blk, axis=0, keepdims=True)
    both = jnp.concatenate([s1, s2], axis=0)

    @pl.when(i == 0)
    def _init():
        acc_ref[...] = both

    @pl.when(i > 0)
    def _acc():
        acc_ref[...] += both


def _ctrl_stats_body(e_ref, s_ref, t_ref, cw_ref, cb_ref, acc_ref):
    i = pl.program_id(0)
    en = e_ref[...] * s_ref[...] + t_ref[...]
    w = jnp.dot(en, cw_ref[...], preferred_element_type=jnp.float32) + cb_ref[...]
    s1 = jnp.sum(w, axis=0, keepdims=True)
    s2 = jnp.sum(w * w, axis=0, keepdims=True)
    both = jnp.concatenate([s1, s2], axis=0)

    @pl.when(i == 0)
    def _init():
        acc_ref[...] = both

    @pl.when(i > 0)
    def _acc():
        acc_ref[...] += both


def _main_body(e_ref, s_ref, t_ref, cw_ref, cb_ref, wa_ref, wb_ref, ex_ref,
               vw_ref, cst_ref, w1_ref, b1_ref, h1_ref, p1_ref, acc_ref):
    i = pl.program_id(0)
    en = e_ref[...] * s_ref[...] + t_ref[...]
    w = jnp.dot(en, cw_ref[...], preferred_element_type=jnp.float32) + cb_ref[...]
    wn = jnp.maximum(w * wa_ref[...] + wb_ref[...], 0.0)
    cnt = jnp.zeros_like(wn)
    for g in range(_F):
        cnt = cnt + (wn[:, g:g + 1] > wn).astype(jnp.float32)
    sel = (cnt < float(_K)) & (wn > 0.0)
    wsel = jnp.where(sel, wn, 0.0)
    mask = wsel / jnp.sum(wsel, axis=1, keepdims=True)
    x0 = en * jnp.dot(mask, ex_ref[...], preferred_element_type=jnp.float32)
    vv = jnp.dot(x0, vw_ref[...], preferred_element_type=jnp.float32)
    a1 = 1.0 + vv[:, 0:1]
    a2 = a1 * (1.0 + vv[:, 1:2]) + cst_ref[0:1, 0:1]
    a3 = a2 * (1.0 + vv[:, 2:3]) + cst_ref[0:1, 1:2]
    p1_ref[...] = a3 * vv[:, 3:4] + cst_ref[0:1, 2:3]
    h1 = jnp.dot(x0, w1_ref[...], preferred_element_type=jnp.float32) + b1_ref[...]
    h1_ref[...] = h1
    s1 = jnp.sum(h1, axis=0, keepdims=True)
    s2 = jnp.sum(h1 * h1, axis=0, keepdims=True)
    both = jnp.concatenate([s1, s2], axis=0)

    @pl.when(i == 0)
    def _init():
        acc_ref[...] = both

    @pl.when(i > 0)
    def _acc():
        acc_ref[...] += both


def _mlp2_body(h1_ref, a_ref, c_ref, w2_ref, b2_ref, h2_ref, acc_ref):
    i = pl.program_id(0)
    h = jnp.maximum(h1_ref[...] * a_ref[...] + c_ref[...], 0.0)
    h2 = jnp.dot(h, w2_ref[...], preferred_element_type=jnp.float32) + b2_ref[...]
    h2_ref[...] = h2
    s1 = jnp.sum(h2, axis=0, keepdims=True)
    s2 = jnp.sum(h2 * h2, axis=0, keepdims=True)
    both = jnp.concatenate([s1, s2], axis=0)

    @pl.when(i == 0)
    def _init():
        acc_ref[...] = both

    @pl.when(i > 0)
    def _acc():
        acc_ref[...] += both


def _final_body(h2_ref, p1_ref, a_ref, c_ref, wl_ref, out_ref):
    h = jnp.maximum(h2_ref[...] * a_ref[...] + c_ref[...], 0.0)
    p = jnp.dot(h, wl_ref[...], preferred_element_type=jnp.float32) + p1_ref[...]
    out_ref[...] = jax.nn.sigmoid(p)


def kernel(x, table, bn_gamma, bn_beta, ctrl_W, ctrl_b, ctrl_bn_g, ctrl_bn_b,
           cross_W, cross_b, mlp_W1, mlp_b1, bn1_g, bn1_b, mlp_W2, mlp_b2,
           bn2_g, bn2_b, lin_W, lin_b):
    idx = (x + (jnp.arange(_F, dtype=jnp.int32) * _VOCAB)[None, :]).reshape(-1)
    idx2 = idx.reshape(_ROWS // _G, _G)
    rows = _gather_kernel(table, idx2)
    e = rows.reshape(_B, _ED)

    stats = pl.pallas_call(
        _field_stats_body,
        grid=(_NB,),
        in_specs=[pl.BlockSpec((_BS, _ED), lambda i: (i, 0))],
        out_specs=pl.BlockSpec((2, _ED), lambda i: (0, 0)),
        out_shape=jax.ShapeDtypeStruct((2, _ED), jnp.float32),
    )(e)
    n = float(_B * _D)
    fsum = stats[0].reshape(_F, _D).sum(axis=1)
    fsq = stats[1].reshape(_F, _D).sum(axis=1)
    fm = fsum / n
    fv = fsq / n - fm * fm
    sf = bn_gamma / jnp.sqrt(fv + _EPS)
    tf = bn_beta - fm * sf
    s_vec = jnp.repeat(sf, _D)[None, :]
    t_vec = jnp.repeat(tf, _D)[None, :]

    const_spec = lambda shape: pl.BlockSpec(shape, lambda i: tuple(0 for _ in shape))
    wstats = pl.pallas_call(
        _ctrl_stats_body,
        grid=(_NB,),
        in_specs=[
            pl.BlockSpec((_BS, _ED), lambda i: (i, 0)),
            const_spec((1, _ED)),
            const_spec((1, _ED)),
            const_spec((_ED, _F)),
            const_spec((1, _F)),
        ],
        out_specs=pl.BlockSpec((2, _F), lambda i: (0, 0)),
        out_shape=jax.ShapeDtypeStruct((2, _F), jnp.float32),
    )(e, s_vec, t_vec, ctrl_W, ctrl_b[None, :])
    wm = wstats[0] / _B
    wv = wstats[1] / _B - wm * wm
    wa = ctrl_bn_g[None, :] / jnp.sqrt(wv + _EPS)
    wb = ctrl_bn_b[None, :] - wm * wa

    ex = (jnp.arange(_ED, dtype=jnp.int32)[None, :] // _D
          == jnp.arange(_F, dtype=jnp.int32)[:, None]).astype(jnp.float32)
    lin_top = lin_W[:_ED, :]
    vw = jnp.concatenate([cross_W.T, lin_top], axis=1)
    c01 = jnp.dot(cross_b[0], cross_W[1])
    c2s = jnp.dot(cross_b[0] + cross_b[1], cross_W[2])
    pconst = jnp.dot(cross_b[0] + cross_b[1] + cross_b[2], lin_top[:, 0]) + lin_b[0]
    cst = jnp.stack([c01, c2s, pconst, jnp.float32(0)])[None, :]

    h1, p1, h1stats = pl.pallas_call(
        _main_body,
        grid=(_NB,),
        in_specs=[
            pl.BlockSpec((_BS, _ED), lambda i: (i, 0)),
            const_spec((1, _ED)),
            const_spec((1, _ED)),
            const_spec((_ED, _F)),
            const_spec((1, _F)),
            const_spec((1, _F)),
            const_spec((1, _F)),
            const_spec((_F, _ED)),
            const_spec((_ED, 4)),
            const_spec((1, 4)),
            const_spec((_ED, 128)),
            const_spec((1, 128)),
        ],
        out_specs=[
            pl.BlockSpec((_BS, 128), lambda i: (i, 0)),
            pl.BlockSpec((_BS, 1), lambda i: (i, 0)),
            pl.BlockSpec((2, 128), lambda i: (0, 0)),
        ],
        out_shape=[
            jax.ShapeDtypeStruct((_B, 128), jnp.float32),
            jax.ShapeDtypeStruct((_B, 1), jnp.float32),
            jax.ShapeDtypeStruct((2, 128), jnp.float32),
        ],
    )(e, s_vec, t_vec, ctrl_W, ctrl_b[None, :], wa, wb, ex, vw, cst,
      mlp_W1, mlp_b1[None, :])
    h1m = h1stats[0] / _B
    h1v = h1stats[1] / _B - h1m * h1m
    a1 = bn1_g[None, :] / jnp.sqrt(h1v + _EPS)
    c1 = bn1_b[None, :] - h1m * a1

    h2, h2stats = pl.pallas_call(
        _mlp2_body,
        grid=(_NB,),
        in_specs=[
            pl.BlockSpec((_BS, 128), lambda i: (i, 0)),
            const_spec((1, 128)),
            const_spec((1, 128)),
            const_spec((128, 64)),
            const_spec((1, 64)),
        ],
        out_specs=[
            pl.BlockSpec((_BS, 64), lambda i: (i, 0)),
            pl.BlockSpec((2, 64), lambda i: (0, 0)),
        ],
        out_shape=[
            jax.ShapeDtypeStruct((_B, 64), jnp.float32),
            jax.ShapeDtypeStruct((2, 64), jnp.float32),
        ],
    )(h1, a1, c1, mlp_W2, mlp_b2[None, :])
    h2m = h2stats[0] / _B
    h2v = h2stats[1] / _B - h2m * h2m
    a2 = bn2_g[None, :] / jnp.sqrt(h2v + _EPS)
    c2 = bn2_b[None, :] - h2m * a2

    out = pl.pallas_call(
        _final_body,
        grid=(_NB,),
        in_specs=[
            pl.BlockSpec((_BS, 64), lambda i: (i, 0)),
            pl.BlockSpec((_BS, 1), lambda i: (i, 0)),
            const_spec((1, 64)),
            const_spec((1, 64)),
            const_spec((64, 1)),
        ],
        out_specs=pl.BlockSpec((_BS, 1), lambda i: (i, 0)),
        out_shape=jax.ShapeDtypeStruct((_B, 1), jnp.float32),
    )(h2, p1, a2, c2, lin_W[_ED:, :])
    return out.reshape(_B)
